# baseline (speedup 1.0000x reference)
import functools

import jax
import jax.numpy as jnp
from jax import lax
from jax.experimental import pallas as pl
from jax.experimental.pallas import tpu as pltpu
from jax.experimental.pallas import tpu_sc as plsc

_NC = 2
_NS = 16
_NW = _NC * _NS
_CH = 104
_NBUF = 4


@functools.lru_cache(maxsize=None)
def _make_gather(B, D):
    n_chunks = B // (_NW * _CH)
    assert n_chunks % _NBUF == 0
    mesh = plsc.VectorSubcoreMesh(
        core_axis_name="c", subcore_axis_name="s",
        num_cores=_NC, num_subcores=_NS)

    @functools.partial(
        pl.kernel,
        out_type=jax.ShapeDtypeStruct((B, D), jnp.float32),
        mesh=mesh,
        compiler_params=pltpu.CompilerParams(
            disable_bounds_checks=True, skip_device_barrier=True),
        scratch_types=[
            pltpu.VMEM((n_chunks, _CH), jnp.int32),
            [pltpu.VMEM((_CH, D), jnp.float32)] * _NBUF,
            [pltpu.SemaphoreType.DMA] * _NBUF,
            [pltpu.SemaphoreType.DMA] * _NBUF,
        ],
    )
    def gather_kernel(table_hbm, idx_hbm, out_hbm, idx_v, bufs,
                      gsems, wsems):
        wid = lax.axis_index("s") * _NC + lax.axis_index("c")
        base = wid * (n_chunks * _CH)
        pltpu.sync_copy(idx_hbm.at[wid], idx_v)

        def start_gather(c, b):
            pltpu.async_copy(table_hbm.at[idx_v.at[c]], bufs[b], gsems[b])

        def wait_gather(b):
            pltpu.make_async_copy(
                table_hbm.at[idx_v.at[0]], bufs[b], gsems[b]).wait()

        def out_slice(c):
            return out_hbm.at[pl.ds(base + c * _CH, _CH)]

        def start_write(c, b):
            pltpu.async_copy(bufs[b], out_slice(c), wsems[b])

        def wait_write(b):
            pltpu.make_async_copy(bufs[b], out_slice(0), wsems[b]).wait()

        for b in range(_NBUF):
            start_gather(b, b)

        skew = _NBUF // 2

        def loop_body(p, carry):
            for b in range(_NBUF):
                c = p * _NBUF + b
                r = c - skew
                rb = (b - skew) % _NBUF

                @pl.when(jnp.logical_and(r >= 0, r + _NBUF < n_chunks))
                def _():
                    wait_write(rb)
                    start_gather(r + _NBUF, rb)

                wait_gather(b)
                start_write(c, b)
            return carry

        lax.fori_loop(0, n_chunks // _NBUF, loop_body, 0)
        for b in range(_NBUF):
            wait_write(b)

    return gather_kernel


def kernel(table, indices):
    N, F = indices.shape
    D = table.shape[1]
    B = N * F
    idx = indices.T.reshape(_NW, B // (_NW * _CH), _CH)
    out = _make_gather(B, D)(table, idx)
    return out.reshape(F, N, D).transpose(1, 0, 2)


# speedup vs baseline: 1.0202x; 1.0202x over previous
import functools

import jax
import jax.numpy as jnp
from jax import lax
from jax.experimental import pallas as pl
from jax.experimental.pallas import tpu as pltpu
from jax.experimental.pallas import tpu_sc as plsc

_NC = 2
_NS = 16
_NW = _NC * _NS
_CH = 104
_NBUF = 8


@functools.lru_cache(maxsize=None)
def _make_gather(B, D):
    n_chunks = B // (_NW * _CH)
    assert n_chunks % _NBUF == 0
    mesh = plsc.VectorSubcoreMesh(
        core_axis_name="c", subcore_axis_name="s",
        num_cores=_NC, num_subcores=_NS)

    @functools.partial(
        pl.kernel,
        out_type=jax.ShapeDtypeStruct((B, D), jnp.float32),
        mesh=mesh,
        compiler_params=pltpu.CompilerParams(
            disable_bounds_checks=True, skip_device_barrier=True),
        scratch_types=[
            pltpu.VMEM((n_chunks, _CH), jnp.int32),
            [pltpu.VMEM((_CH, D), jnp.float32)] * _NBUF,
            [pltpu.SemaphoreType.DMA] * _NBUF,
            [pltpu.SemaphoreType.DMA] * _NBUF,
        ],
    )
    def gather_kernel(table_hbm, idx_hbm, out_hbm, idx_v, bufs,
                      gsems, wsems):
        wid = lax.axis_index("s") * _NC + lax.axis_index("c")
        base = wid * (n_chunks * _CH)
        pltpu.sync_copy(idx_hbm.at[wid], idx_v)

        def start_gather(c, b):
            pltpu.async_copy(table_hbm.at[idx_v.at[c]], bufs[b], gsems[b])

        def wait_gather(b):
            pltpu.make_async_copy(
                table_hbm.at[idx_v.at[0]], bufs[b], gsems[b]).wait()

        def out_slice(c):
            return out_hbm.at[pl.ds(base + c * _CH, _CH)]

        def start_write(c, b):
            pltpu.async_copy(bufs[b], out_slice(c), wsems[b])

        def wait_write(b):
            pltpu.make_async_copy(bufs[b], out_slice(0), wsems[b]).wait()

        for b in range(_NBUF):
            start_gather(b, b)

        skew = 2

        def loop_body(p, carry):
            for b in range(_NBUF):
                c = p * _NBUF + b
                r = c - skew
                rb = (b - skew) % _NBUF

                @pl.when(jnp.logical_and(r >= 0, r + _NBUF < n_chunks))
                def _():
                    wait_write(rb)
                    start_gather(r + _NBUF, rb)

                wait_gather(b)
                start_write(c, b)
            return carry

        lax.fori_loop(0, n_chunks // _NBUF, loop_body, 0)
        for b in range(_NBUF):
            wait_write(b)

    return gather_kernel


def kernel(table, indices):
    N, F = indices.shape
    D = table.shape[1]
    B = N * F
    idx = indices.T.reshape(_NW, B // (_NW * _CH), _CH)
    out = _make_gather(B, D)(table, idx)
    return out.reshape(F, N, D).transpose(1, 0, 2)


# speedup vs baseline: 1.0409x; 1.0202x over previous
import functools

import jax
import jax.numpy as jnp
from jax import lax
from jax.experimental import pallas as pl
from jax.experimental.pallas import tpu as pltpu
from jax.experimental.pallas import tpu_sc as plsc

_NC = 2
_NS = 16
_NW = _NC * _NS
_NBUF = 6
_SKEW = 3


@functools.lru_cache(maxsize=None)
def _make_gather(N, F, D):
    bpw = N // _NW
    n_chunks = F
    n_passes = -(-n_chunks // _NBUF)
    mesh = plsc.VectorSubcoreMesh(
        core_axis_name="c", subcore_axis_name="s",
        num_cores=_NC, num_subcores=_NS)

    @functools.partial(
        pl.kernel,
        out_type=jax.ShapeDtypeStruct((N * F, D), jnp.float32),
        mesh=mesh,
        compiler_params=pltpu.CompilerParams(
            disable_bounds_checks=True, skip_device_barrier=True),
        scratch_types=[
            pltpu.VMEM((F, bpw), jnp.int32),
            [pltpu.VMEM((bpw, D), jnp.float32)] * _NBUF,
            [pltpu.SemaphoreType.DMA] * _NBUF,
            [pltpu.SemaphoreType.DMA] * _NBUF,
        ],
    )
    def gather_kernel(table_hbm, idx_hbm, out_hbm, idx_v, bufs,
                      gsems, wsems):
        wid = lax.axis_index("s") * _NC + lax.axis_index("c")
        b0 = wid * bpw
        pltpu.sync_copy(idx_hbm.at[:, pl.ds(b0, bpw)], idx_v)

        def start_gather(c, b):
            pltpu.async_copy(table_hbm.at[idx_v.at[c]], bufs[b], gsems[b])

        def wait_gather(b):
            pltpu.make_async_copy(
                table_hbm.at[idx_v.at[0]], bufs[b], gsems[b]).wait()

        def out_slice(c):
            return out_hbm.at[pl.ds(c * N + b0, bpw)]

        def start_write(c, b):
            pltpu.async_copy(bufs[b], out_slice(c), wsems[b])

        def wait_write(b):
            pltpu.make_async_copy(bufs[b], out_slice(0), wsems[b]).wait()

        for b in range(_NBUF):
            start_gather(b, b)

        def chunk_step(c, b):
            r = c - _SKEW
            rb = (b - _SKEW) % _NBUF

            @pl.when(jnp.logical_and(r >= 0, r + _NBUF < n_chunks))
            def _():
                wait_write(rb)
                start_gather(r + _NBUF, rb)

            @pl.when(c < n_chunks)
            def _():
                wait_gather(b)
                start_write(c, b)

        def loop_body(p, carry):
            for b in range(_NBUF):
                chunk_step(p * _NBUF + b, b)
            return carry

        lax.fori_loop(0, n_passes, loop_body, 0)
        for b in range(_NBUF):
            wait_write(b)

    return gather_kernel


def kernel(table, indices):
    N, F = indices.shape
    D = table.shape[1]
    out = _make_gather(N, F, D)(table, indices.T)
    return out.reshape(F, N, D).transpose(1, 0, 2)
